# Initial kernel scaffold; baseline (speedup 1.0000x reference)
import jax
import jax.numpy as jnp
from jax.experimental import pallas as pl


def kernel(x, edge_index, batch, W1_l, W1_r, b1, W2_l, W2_r, b2):
    raise NotImplementedError("write your pallas kernel here")


# speedup vs baseline: 5.5568x; 5.5568x over previous
import functools

import jax
import jax.numpy as jnp
from jax import lax
from jax.experimental import pallas as pl
from jax.experimental.pallas import tpu as pltpu
from jax.experimental.pallas import tpu_sc as plsc

_N = 10000
_E = 320000
_D = 128
_G = 64
_WAUG = 144

_K = 80
_NC = 2
_NS = 16
_NW = _NC * _NS
_EPW = _E // _NW
_NCHUNK = _EPW // _K
_RPT = _N // _NS
_ZR = 125
_NZ = _RPT // _ZR

_BN = 1000
_PREC = lax.Precision.HIGHEST


def _make_sc_pass(width):
  mesh = plsc.VectorSubcoreMesh(core_axis_name="c", subcore_axis_name="s")

  def body(table_hbm, gidx_hbm, sidx_hbm, z_hbm, out_hbm,
           gv, sv, rows, zb, acc_sh, sem):
    c = lax.axis_index("c")
    s = lax.axis_index("s")
    w = s * _NC + c

    pltpu.sync_copy(z_hbm, zb)

    def zstep(k, carry):
      pltpu.sync_copy(zb, acc_sh.at[pl.ds(s * _RPT + k * _ZR, _ZR)])
      return carry
    lax.fori_loop(0, _NZ, zstep, 0)
    plsc.subcore_barrier()

    rbase = w * _NCHUNK

    def estep(i, carry):
      pltpu.sync_copy(gidx_hbm.at[rbase + i], gv)
      pltpu.sync_copy(sidx_hbm.at[rbase + i], sv)
      pltpu.async_copy(table_hbm.at[gv], rows, sem).wait()
      pltpu.sync_copy(rows, acc_sh.at[sv], add=True)
      return carry
    lax.fori_loop(0, _NCHUNK, estep, 0)
    plsc.subcore_barrier()

    def ostep(k, carry):
      r0 = s * _RPT + k * _ZR
      pltpu.sync_copy(acc_sh.at[pl.ds(r0, _ZR)], zb)
      pltpu.sync_copy(zb, out_hbm.at[c, pl.ds(r0, _ZR)])
      return carry
    lax.fori_loop(0, _NZ, ostep, 0)

  return pl.kernel(
      body,
      out_type=jax.ShapeDtypeStruct((_NC, _N, width), jnp.float32),
      mesh=mesh,
      compiler_params=pltpu.CompilerParams(use_tc_tiling_on_sc=False),
      scratch_types=[
          pltpu.VMEM((_K,), jnp.int32),
          pltpu.VMEM((_K,), jnp.int32),
          pltpu.VMEM((_K, width), jnp.float32),
          pltpu.VMEM((_ZR, width), jnp.float32),
          pltpu.VMEM_SHARED((_N, width), jnp.float32),
          pltpu.SemaphoreType.DMA,
      ],
  )


_sc_pass_a = _make_sc_pass(_WAUG)
_sc_pass_b = _make_sc_pass(_G)


def _prep_body(batch_ref, d0_ref, d1_ref, scale_ref, q_ref, cmat_ref, gnz_ref):
  gi = lax.broadcasted_iota(jnp.int32, (1, _G), 1)
  oh = (batch_ref[...] == gi).astype(jnp.float32)
  gc = jnp.sum(oh, axis=0, keepdims=True)
  gcc = jnp.maximum(gc, 1.0)
  deg = d0_ref[...] + d1_ref[...]
  scale = 1.0 / jnp.maximum(deg, 1.0)
  ohg = oh / gcc
  scale_ref[...] = scale
  cmat_ref[...] = ohg
  q_ref[...] = ohg * scale
  gnz_ref[...] = jnp.minimum(gc, 1.0)


_prep = pl.pallas_call(
    _prep_body,
    out_shape=(
        jax.ShapeDtypeStruct((_N, 1), jnp.float32),
        jax.ShapeDtypeStruct((_N, _G), jnp.float32),
        jax.ShapeDtypeStruct((_N, _G), jnp.float32),
        jax.ShapeDtypeStruct((1, _G), jnp.float32),
    ),
)


def _h_body(p0, p1, x, scale, w1l, w1r, b1, h_out):
  mean = (p0[...] + p1[...]) * scale[...]
  acc = jnp.dot(mean, w1l[...], preferred_element_type=jnp.float32,
                precision=_PREC)
  acc = acc + jnp.dot(x[...], w1r[...], preferred_element_type=jnp.float32,
                      precision=_PREC)
  h_out[...] = jnp.maximum(acc + b1[...], 0.0)


_h_call = pl.pallas_call(
    _h_body,
    grid=(_N // _BN,),
    in_specs=[
        pl.BlockSpec((_BN, _D), lambda i: (i, 0)),
        pl.BlockSpec((_BN, _D), lambda i: (i, 0)),
        pl.BlockSpec((_BN, _D), lambda i: (i, 0)),
        pl.BlockSpec((_BN, 1), lambda i: (i, 0)),
        pl.BlockSpec((_D, _D), lambda i: (0, 0)),
        pl.BlockSpec((_D, _D), lambda i: (0, 0)),
        pl.BlockSpec((1, _D), lambda i: (0, 0)),
    ],
    out_specs=pl.BlockSpec((_BN, _D), lambda i: (i, 0)),
    out_shape=jax.ShapeDtypeStruct((_N, _D), jnp.float32),
)


def _final_body(s0, s1, cmat, h, w2l, w2r, b2, gnz, out_ref, acc):
  i = pl.program_id(0)

  @pl.when(i == 0)
  def _init():
    acc[...] = jnp.zeros_like(acc)

  u = jnp.concatenate([s0[...] + s1[...], cmat[...]], axis=1)
  acc[...] += lax.dot_general(
      u, h[...], (((0,), (0,)), ((), ())),
      preferred_element_type=jnp.float32, precision=_PREC)

  @pl.when(i == pl.num_programs(0) - 1)
  def _fin():
    t = acc[...]
    out_ref[...] = (
        jnp.dot(t[:_G], w2l[...], preferred_element_type=jnp.float32,
                precision=_PREC)
        + jnp.dot(t[_G:], w2r[...], preferred_element_type=jnp.float32,
                  precision=_PREC)
        + gnz[...].T * b2[...])


_final_call = pl.pallas_call(
    _final_body,
    grid=(_N // _BN,),
    in_specs=[
        pl.BlockSpec((_BN, _G), lambda i: (i, 0)),
        pl.BlockSpec((_BN, _G), lambda i: (i, 0)),
        pl.BlockSpec((_BN, _G), lambda i: (i, 0)),
        pl.BlockSpec((_BN, _D), lambda i: (i, 0)),
        pl.BlockSpec((_D, _D), lambda i: (0, 0)),
        pl.BlockSpec((_D, _D), lambda i: (0, 0)),
        pl.BlockSpec((1, _D), lambda i: (0, 0)),
        pl.BlockSpec((1, _G), lambda i: (0, 0)),
    ],
    out_specs=pl.BlockSpec((_G, _D), lambda i: (0, 0)),
    out_shape=jax.ShapeDtypeStruct((_G, _D), jnp.float32),
    scratch_shapes=[pltpu.VMEM((2 * _G, _D), jnp.float32)],
)


def kernel(x, edge_index, batch, W1_l, W1_r, b1, W2_l, W2_r, b2):
  x = x.astype(jnp.float32)
  src = edge_index[0].reshape(_E // _K, _K)
  dst = edge_index[1].reshape(_E // _K, _K)
  pad = jnp.concatenate(
      [jnp.ones((_N, 1), jnp.float32),
       jnp.zeros((_N, _WAUG - _D - 1), jnp.float32)], axis=1)
  x_aug = jnp.concatenate([x, pad], axis=1)
  z_a = jnp.zeros((_ZR, _WAUG), jnp.float32)
  z_b = jnp.zeros((_ZR, _G), jnp.float32)

  agg = _sc_pass_a(x_aug, src, dst, z_a)
  p0 = agg[0, :, :_D]
  p1 = agg[1, :, :_D]
  d0 = agg[0, :, _D:_D + 1]
  d1 = agg[1, :, _D:_D + 1]

  scale, q, cmat, gnz = _prep(batch.reshape(_N, 1), d0, d1)
  h = _h_call(p0, p1, x, scale, W1_l, W1_r, b1.reshape(1, _D))
  spart = _sc_pass_b(q, dst, src, z_b)
  return _final_call(spart[0], spart[1], cmat, h,
                     W2_l, W2_r, b2.reshape(1, _D), gnz)
